# baseline (speedup 1.0000x reference)
import functools

import jax
import jax.numpy as jnp
from jax import lax
from jax.experimental import pallas as pl
from jax.experimental.pallas import tpu as pltpu
from jax.experimental.pallas import tpu_sc as plsc

N_TRANS = 10000
N_MACRO = 1000
E_TRANS = 320000
E_MACRO = 16000
TRANS_IN = 128
MACRO_IN = 32
HID = 128

NC, NS = 2, 16
NW = NC * NS
CHUNK = 128
HALF = TRANS_IN // 2

D_T = 80
D_M = 80
ACC_T = 10112
ACC_M = 1024

T_CHUNKS = -(-E_TRANS // (NS * CHUNK))
M_CHUNKS = -(-E_MACRO // (NW * CHUNK))
E_T_PAD = T_CHUNKS * NS * CHUNK
E_M_PAD = M_CHUNKS * NW * CHUNK

NBUF = 4
H0 = (T_CHUNKS + 1) // 2
H1 = T_CHUNKS - H0


def _sc_aggregate(t_tab0, t_tab1, m_tab, t_src, t_dst, m_src, m_dst, zt, zm):
    mesh = plsc.VectorSubcoreMesh(core_axis_name="c", subcore_axis_name="s")

    @functools.partial(
        pl.kernel,
        out_type=(
            jax.ShapeDtypeStruct((NC, ACC_T, D_T), jnp.float32),
            jax.ShapeDtypeStruct((NC, ACC_M, D_M), jnp.float32),
        ),
        mesh=mesh,
        scratch_types=[
            pltpu.VMEM_SHARED((ACC_T, D_T), jnp.float32),
            pltpu.VMEM_SHARED((ACC_M, D_M), jnp.float32),
            pltpu.VMEM((H0, CHUNK), jnp.int32),
            pltpu.VMEM((H0, CHUNK), jnp.int32),
            pltpu.VMEM((M_CHUNKS, CHUNK), jnp.int32),
            pltpu.VMEM((M_CHUNKS, CHUNK), jnp.int32),
            pltpu.VMEM((NBUF, CHUNK, D_T), jnp.float32),
            pltpu.SemaphoreType.DMA,
            pltpu.SemaphoreType.DMA,
        ],
        compiler_params=pltpu.CompilerParams(use_tc_tiling_on_sc=False),
    )
    def k(t0_hbm, t1_hbm, m_hbm, tsrc_hbm, tdst_hbm, msrc_hbm, mdst_hbm,
          zt_hbm, zm_hbm, tpart_hbm, mpart_hbm,
          acc_t, acc_m, sidx, didx, msidx, mdidx, rows, sem_g, sem_s):
        cid = lax.axis_index("c")
        sid = lax.axis_index("s")
        wid = sid * NC + cid
        base = sid * T_CHUNKS

        pf = [pltpu.async_copy(tsrc_hbm.at[pl.ds(base, H0)], sidx, sem_g),
              pltpu.async_copy(tdst_hbm.at[pl.ds(base, H0)], didx, sem_g)]

        rt = ACC_T // NS
        rm = ACC_M // NS
        pltpu.sync_copy(zt_hbm, acc_t.at[pl.ds(sid * rt, rt)])
        pltpu.sync_copy(zm_hbm, acc_m.at[pl.ds(sid * rm, rm)])
        for d in pf:
            d.wait()
        plsc.subcore_barrier()

        def run_group(x_hbm, src2d, dst2d, acc, j0, n):
            gd = [pltpu.async_copy(x_hbm.at[src2d.at[j0 + b]], rows.at[b], sem_g)
                  for b in range(n)]
            for d in gd:
                d.wait()
            sd = [pltpu.async_copy(rows.at[b], acc.at[dst2d.at[j0 + b]], sem_s,
                                   add=True)
                  for b in range(n)]
            for d in sd:
                d.wait()

        G2 = NBUF // 2

        def trans_loop(tab_hbm):
            def fire_g(j0, s):
                for b in range(G2):
                    pltpu.async_copy(tab_hbm.at[sidx.at[j0 + b]],
                                     rows.at[s * G2 + b], sem_g)

            def drain_g(s):
                for b in range(G2):
                    pltpu.make_async_copy(tab_hbm.at[sidx.at[0]],
                                          rows.at[s * G2 + b], sem_g).wait()

            def fire_s(j0, s):
                for b in range(G2):
                    pltpu.async_copy(rows.at[s * G2 + b],
                                     acc_t.at[didx.at[j0 + b]], sem_s, add=True)

            def drain_s(s):
                for b in range(G2):
                    pltpu.make_async_copy(rows.at[s * G2 + b],
                                          acc_t.at[didx.at[0]], sem_s).wait()

            for h, nch in ((0, H0), (1, H1)):
                if h:
                    pltpu.sync_copy(tsrc_hbm.at[pl.ds(base + h * H0, nch)],
                                    sidx.at[pl.ds(0, nch)])
                    pltpu.sync_copy(tdst_hbm.at[pl.ds(base + h * H0, nch)],
                                    didx.at[pl.ds(0, nch)])
                ng, tail = divmod(nch, G2)
                assert ng % 2 == 1

                fire_g(0, 0)

                def pair(p, _):
                    fire_g((2 * p + 1) * G2, 1)
                    drain_g(0)
                    fire_s(2 * p * G2, 0)
                    drain_s(0)
                    fire_g((2 * p + 2) * G2, 0)
                    drain_g(1)
                    fire_s((2 * p + 1) * G2, 1)
                    drain_s(1)
                    return _

                lax.fori_loop(0, (ng - 1) // 2, pair, None)
                drain_g(0)
                fire_s((ng - 1) * G2, 0)
                drain_s(0)
                if tail:
                    run_group(tab_hbm, sidx, didx, acc_t, ng * G2, tail)

        @pl.when(cid == 0)
        def _():
            trans_loop(t0_hbm)

        @pl.when(cid == 1)
        def _():
            trans_loop(t1_hbm)

        pltpu.sync_copy(msrc_hbm.at[pl.ds(wid * M_CHUNKS, M_CHUNKS)], msidx)
        pltpu.sync_copy(mdst_hbm.at[pl.ds(wid * M_CHUNKS, M_CHUNKS)], mdidx)
        run_group(m_hbm, msidx, mdidx, acc_m, 0, M_CHUNKS)

        plsc.subcore_barrier()

        pltpu.sync_copy(acc_t.at[pl.ds(sid * rt, rt)],
                        tpart_hbm.at[cid, pl.ds(sid * rt, rt)])
        pltpu.sync_copy(acc_m.at[pl.ds(sid * rm, rm)],
                        mpart_hbm.at[cid, pl.ds(sid * rm, rm)])

    return k(t_tab0, t_tab1, m_tab, t_src, t_dst, m_src, m_dst, zt, zm)


BLK = 2000
GRID = N_TRANS // BLK


def _tc_dense_kernel(tx_ref, tpart_ref, n_ref, mx_ref, mpart_ref,
                     wms_ref, wmnl_ref, wmnh_ref, bmi_ref,
                     wMs_ref, wMn_ref, bma_ref,
                     wct_ref, wcb_ref, bct_ref, wp_ref, bp_ref,
                     out_ref, g_scr):
    i = pl.program_id(0)

    @pl.when(i == 0)
    def _():
        ms = mpart_ref[0, :N_MACRO, :MACRO_IN] + mpart_ref[1, :N_MACRO, :MACRO_IN]
        mc = mpart_ref[0, :N_MACRO, MACRO_IN:MACRO_IN + 1] + \
             mpart_ref[1, :N_MACRO, MACRO_IN:MACRO_IN + 1]
        m_agg = ms / jnp.maximum(mc, 1.0)
        h_macro = jnp.maximum(
            jnp.dot(mx_ref[...], wMs_ref[...], preferred_element_type=jnp.float32)
            + jnp.dot(m_agg, wMn_ref[...], preferred_element_type=jnp.float32)
            + bma_ref[...], 0.0)
        g_scr[...] = jnp.dot(h_macro, wcb_ref[...], preferred_element_type=jnp.float32)

    tc = jnp.maximum(tpart_ref[0, :, HALF:HALF + 1], 1.0)
    t_agg_lo = tpart_ref[0, :, :HALF] / tc
    t_agg_hi = tpart_ref[1, :, :HALF] / tc
    h_micro = jnp.maximum(
        jnp.dot(tx_ref[...], wms_ref[...], preferred_element_type=jnp.float32)
        + jnp.dot(t_agg_lo, wmnl_ref[...], preferred_element_type=jnp.float32)
        + jnp.dot(t_agg_hi, wmnh_ref[...], preferred_element_type=jnp.float32)
        + bmi_ref[...], 0.0)

    cols = lax.broadcasted_iota(jnp.int32, (BLK, N_MACRO), 1)
    onehot = (n_ref[...] == cols).astype(jnp.float32)
    macro_per_trans = jnp.dot(onehot, g_scr[...], preferred_element_type=jnp.float32)

    h_final = jnp.maximum(
        jnp.dot(h_micro, wct_ref[...], preferred_element_type=jnp.float32)
        + macro_per_trans + bct_ref[...], 0.0)
    out_ref[...] = jnp.dot(h_final, wp_ref[...],
                           preferred_element_type=jnp.float32) + bp_ref[...]


def _tc_dense(tx, tpart, n2d, mx, mpart, wms, wmnl, wmnh, bmi, wMs, wMn, bma,
              wct, wcb, bct, wp, bp):
    whole = lambda shape: pl.BlockSpec(shape, lambda i: tuple(0 for _ in shape))
    return pl.pallas_call(
        _tc_dense_kernel,
        grid=(GRID,),
        in_specs=[
            pl.BlockSpec((BLK, TRANS_IN), lambda i: (i, 0)),
            pl.BlockSpec((NC, BLK, D_T), lambda i: (0, i, 0)),
            pl.BlockSpec((BLK, 1), lambda i: (i, 0)),
            whole((N_MACRO, MACRO_IN)),
            whole((NC, ACC_M, D_M)),
            whole((TRANS_IN, HID)),
            whole((HALF, HID)),
            whole((HALF, HID)),
            whole((1, HID)),
            whole((MACRO_IN, HID)),
            whole((MACRO_IN, HID)),
            whole((1, HID)),
            whole((HID, HID)),
            whole((HID, HID)),
            whole((1, HID)),
            whole((HID, 1)),
            whole((1, 1)),
        ],
        out_specs=pl.BlockSpec((BLK, 1), lambda i: (i, 0)),
        out_shape=jax.ShapeDtypeStruct((N_TRANS, 1), jnp.float32),
        scratch_shapes=[pltpu.VMEM((N_MACRO, HID), jnp.float32)],
    )(tx, tpart, n2d, mx, mpart, wms, wmnl, wmnh, bmi, wMs, wMn, bma,
      wct, wcb, bct, wp, bp)


def kernel(trans_x, macro_x, trans_edge_index, macro_edge_index, trans_to_neigh,
           W_micro_self, W_micro_neigh, b_micro,
           W_macro_self, W_macro_neigh, b_macro,
           W_cross_t, b_cross_t, W_cross_m, b_cross_m,
           W_pred, b_pred):
    f32 = jnp.float32

    t_tab0 = jnp.concatenate(
        [trans_x[:, :HALF], jnp.ones((N_TRANS, 1), f32),
         jnp.zeros((N_TRANS, D_T - HALF - 1), f32)], axis=1)
    t_tab1 = jnp.concatenate(
        [trans_x[:, HALF:], jnp.zeros((N_TRANS, D_T - HALF), f32)], axis=1)
    m_tab = jnp.concatenate(
        [macro_x, jnp.ones((N_MACRO, 1), f32),
         jnp.zeros((N_MACRO, D_M - MACRO_IN - 1), f32)], axis=1)

    t_pad = E_T_PAD - E_TRANS
    m_pad = E_M_PAD - E_MACRO
    t_src = jnp.concatenate([trans_edge_index[0], jnp.zeros((t_pad,), jnp.int32)]
                            ).reshape(NS * T_CHUNKS, CHUNK)
    t_dst = jnp.concatenate([trans_edge_index[1], jnp.full((t_pad,), N_TRANS, jnp.int32)]
                            ).reshape(NS * T_CHUNKS, CHUNK)
    m_src = jnp.concatenate([macro_edge_index[0], jnp.zeros((m_pad,), jnp.int32)]
                            ).reshape(NW * M_CHUNKS, CHUNK)
    m_dst = jnp.concatenate([macro_edge_index[1], jnp.full((m_pad,), N_MACRO, jnp.int32)]
                            ).reshape(NW * M_CHUNKS, CHUNK)

    zt = jnp.zeros((ACC_T // NS, D_T), f32)
    zm = jnp.zeros((ACC_M // NS, D_M), f32)

    tpart, mpart = _sc_aggregate(t_tab0, t_tab1, m_tab, t_src, t_dst,
                                 m_src, m_dst, zt, zm)

    out2d = _tc_dense(
        trans_x, tpart, trans_to_neigh.reshape(N_TRANS, 1), macro_x, mpart,
        W_micro_self, W_micro_neigh[:HALF], W_micro_neigh[HALF:],
        b_micro.reshape(1, HID),
        W_macro_self, W_macro_neigh, b_macro.reshape(1, HID),
        W_cross_t[:HID], W_cross_t[HID:], b_cross_t.reshape(1, HID),
        W_pred, b_pred.reshape(1, 1))
    return out2d.squeeze(-1)


# speedup vs baseline: 1.1998x; 1.1998x over previous
import functools

import jax
import jax.numpy as jnp
from jax import lax
from jax.experimental import pallas as pl
from jax.experimental.pallas import tpu as pltpu
from jax.experimental.pallas import tpu_sc as plsc

N_TRANS = 10000
N_MACRO = 1000
E_TRANS = 320000
E_MACRO = 16000
TRANS_IN = 128
MACRO_IN = 32
HID = 128

NC, NS = 2, 16
NW = NC * NS
CHUNK = 128
HALF = TRANS_IN // 2

D_T = 64
D_M = 32
ACC_T = 10240
ACC_M = 1024

T_CHUNKS = -(-E_TRANS // (NS * CHUNK))
M_CHUNKS = -(-E_MACRO // (NS * CHUNK))
E_T_PAD = T_CHUNKS * NS * CHUNK
E_M_PAD = M_CHUNKS * NS * CHUNK

NBUF = 4
H0 = (T_CHUNKS + 1) // 2
H1 = T_CHUNKS - H0


def _sc_aggregate(t_tab0, t_tab1, m_tab, t_src, t_dst, m_src, m_dst, zt, zm):
    mesh = plsc.VectorSubcoreMesh(core_axis_name="c", subcore_axis_name="s")

    @functools.partial(
        pl.kernel,
        out_type=(
            jax.ShapeDtypeStruct((NC, ACC_T, D_T), jnp.float32),
            jax.ShapeDtypeStruct((NC, ACC_M, D_M), jnp.float32),
            jax.ShapeDtypeStruct((ACC_T,), jnp.float32),
            jax.ShapeDtypeStruct((ACC_M,), jnp.float32),
        ),
        mesh=mesh,
        scratch_types=[
            pltpu.VMEM_SHARED((ACC_T, D_T), jnp.float32),
            pltpu.VMEM_SHARED((ACC_M, D_M), jnp.float32),
            pltpu.VMEM_SHARED((NS, ACC_T), jnp.float32),
            pltpu.VMEM_SHARED((NS, ACC_M), jnp.float32),
            pltpu.VMEM((H0, CHUNK), jnp.int32),
            pltpu.VMEM((H0, CHUNK), jnp.int32),
            pltpu.VMEM((M_CHUNKS, CHUNK), jnp.int32),
            pltpu.VMEM((M_CHUNKS, CHUNK), jnp.int32),
            pltpu.VMEM((NBUF, CHUNK, D_T), jnp.float32),
            pltpu.VMEM((2, CHUNK, D_M), jnp.float32),
            pltpu.VMEM((ACC_T,), jnp.float32),
            pltpu.VMEM((ACC_M,), jnp.float32),
            pltpu.VMEM((ACC_T // NS,), jnp.float32),
            pltpu.VMEM((ACC_T // NS,), jnp.float32),
            pltpu.SemaphoreType.DMA,
            pltpu.SemaphoreType.DMA,
        ],
        compiler_params=pltpu.CompilerParams(use_tc_tiling_on_sc=False,
                                             needs_layout_passes=False),
    )
    def k(t0_hbm, t1_hbm, m_hbm, tsrc_hbm, tdst_hbm, msrc_hbm, mdst_hbm,
          zt_hbm, zm_hbm, tpart_hbm, mpart_hbm, cntt_hbm, cntm_hbm,
          acc_t, acc_m, acc_c, acc_mc, sidx, didx, msidx, mdidx, rows, rows_m,
          hist, mhist, mbuf, macc, sem_g, sem_s):
        cid = lax.axis_index("c")
        sid = lax.axis_index("s")
        wid = sid * NC + cid
        base = sid * T_CHUNKS
        ones16 = jnp.full((16,), 1.0, jnp.float32)
        zeros16 = jnp.zeros((16,), jnp.float32)

        def hist_chunk(j):
            for kk in range(CHUNK // 16):
                iv = didx[j, pl.ds(kk * 16, 16)]
                plsc.addupdate_scatter(hist, [iv], ones16)

        pf = [pltpu.async_copy(tsrc_hbm.at[pl.ds(base, H0)], sidx, sem_g),
              pltpu.async_copy(tdst_hbm.at[pl.ds(base, H0)], didx, sem_g)]

        rt = ACC_T // NS
        rm = ACC_M // NS
        pltpu.sync_copy(zt_hbm, acc_t.at[pl.ds(sid * rt, rt)])
        pltpu.sync_copy(zm_hbm, acc_m.at[pl.ds(sid * rm, rm)])
        for d in pf:
            d.wait()
        plsc.subcore_barrier()

        def run_group(x_hbm, rws, src2d, dst2d, acc, j0, n):
            gd = [pltpu.async_copy(x_hbm.at[src2d.at[j0 + b]], rws.at[b], sem_g)
                  for b in range(n)]
            for d in gd:
                d.wait()
            sd = [pltpu.async_copy(rws.at[b], acc.at[dst2d.at[j0 + b]], sem_s,
                                   add=True)
                  for b in range(n)]
            for d in sd:
                d.wait()

        G2 = NBUF // 2

        def trans_loop(tab_hbm, do_hist):
            def fire_g(j0, s):
                for b in range(G2):
                    pltpu.async_copy(tab_hbm.at[sidx.at[j0 + b]],
                                     rows.at[s * G2 + b], sem_g)

            def drain_g(s):
                for b in range(G2):
                    pltpu.make_async_copy(tab_hbm.at[sidx.at[0]],
                                          rows.at[s * G2 + b], sem_g).wait()

            def fire_s(j0, s):
                for b in range(G2):
                    pltpu.async_copy(rows.at[s * G2 + b],
                                     acc_t.at[didx.at[j0 + b]], sem_s, add=True)

            def drain_s(s):
                for b in range(G2):
                    pltpu.make_async_copy(rows.at[s * G2 + b],
                                          acc_t.at[didx.at[0]], sem_s).wait()

            for h, nch in ((0, H0), (1, H1)):
                if h:
                    pltpu.sync_copy(tsrc_hbm.at[pl.ds(base + h * H0, nch)],
                                    sidx.at[pl.ds(0, nch)])
                    pltpu.sync_copy(tdst_hbm.at[pl.ds(base + h * H0, nch)],
                                    didx.at[pl.ds(0, nch)])
                ng, tail = divmod(nch, G2)
                assert ng % 2 == 1

                fire_g(0, 0)

                def pair(p, _):
                    fire_g((2 * p + 1) * G2, 1)
                    drain_g(0)
                    fire_s(2 * p * G2, 0)
                    if do_hist:
                        for b in range(G2):
                            hist_chunk(2 * p * G2 + b)
                    drain_s(0)
                    fire_g((2 * p + 2) * G2, 0)
                    drain_g(1)
                    fire_s((2 * p + 1) * G2, 1)
                    if do_hist:
                        for b in range(G2):
                            hist_chunk((2 * p + 1) * G2 + b)
                    drain_s(1)
                    return _

                lax.fori_loop(0, (ng - 1) // 2, pair, None)
                drain_g(0)
                fire_s((ng - 1) * G2, 0)
                if do_hist:
                    for b in range(G2):
                        hist_chunk((ng - 1) * G2 + b)
                drain_s(0)
                if tail:
                    run_group(tab_hbm, rows, sidx, didx, acc_t, ng * G2, tail)
                    if do_hist:
                        for b in range(tail):
                            hist_chunk(ng * G2 + b)

        @pl.when(cid == 0)
        def _():
            def hz(i, _):
                hist[pl.ds(i * 16, 16)] = zeros16
                return _
            lax.fori_loop(0, ACC_T // 16, hz, None)
            trans_loop(t0_hbm, True)
            pltpu.sync_copy(hist, acc_c.at[sid])

        @pl.when(cid == 1)
        def _():
            trans_loop(t1_hbm, False)

        @pl.when(cid == 0)
        def _():
            pltpu.sync_copy(msrc_hbm.at[pl.ds(sid * M_CHUNKS, M_CHUNKS)], msidx)
            pltpu.sync_copy(mdst_hbm.at[pl.ds(sid * M_CHUNKS, M_CHUNKS)], mdidx)

            def mz(i, _):
                mhist[pl.ds(i * 16, 16)] = zeros16
                return _
            lax.fori_loop(0, ACC_M // 16, mz, None)
            for j in range(M_CHUNKS):
                for kk in range(CHUNK // 16):
                    iv = mdidx[j, pl.ds(kk * 16, 16)]
                    plsc.addupdate_scatter(mhist, [iv], ones16)
            pltpu.sync_copy(mhist, acc_mc.at[sid])

            for g0 in range(0, M_CHUNKS, 2):
                run_group(m_hbm, rows_m, msidx, mdidx, acc_m, g0, 2)

        plsc.subcore_barrier()

        pltpu.sync_copy(acc_t.at[pl.ds(sid * rt, rt)],
                        tpart_hbm.at[cid, pl.ds(sid * rt, rt)])
        pltpu.sync_copy(acc_m.at[pl.ds(sid * rm, rm)],
                        mpart_hbm.at[cid, pl.ds(sid * rm, rm)])

        @pl.when(cid == 0)
        def _():
            pltpu.sync_copy(acc_c.at[0, pl.ds(sid * rt, rt)], macc)

            def addv(i, _):
                sl = pl.ds(i * 16, 16)
                macc[sl] = macc[sl] + mbuf[sl]
                return _

            for t in range(1, NS):
                pltpu.sync_copy(acc_c.at[t, pl.ds(sid * rt, rt)], mbuf)
                lax.fori_loop(0, rt // 16, addv, None)
            pltpu.sync_copy(macc, cntt_hbm.at[pl.ds(sid * rt, rt)])

            pltpu.sync_copy(acc_mc.at[0, pl.ds(sid * rm, rm)],
                            macc.at[pl.ds(0, rm)])

            def addm(i, _):
                sl = pl.ds(i * 16, 16)
                macc[sl] = macc[sl] + mbuf[sl]
                return _

            for t in range(1, NS):
                pltpu.sync_copy(acc_mc.at[t, pl.ds(sid * rm, rm)],
                                mbuf.at[pl.ds(0, rm)])
                lax.fori_loop(0, rm // 16, addm, None)
            pltpu.sync_copy(macc.at[pl.ds(0, rm)],
                            cntm_hbm.at[pl.ds(sid * rm, rm)])

    return k(t_tab0, t_tab1, m_tab, t_src, t_dst, m_src, m_dst, zt, zm)


BLK = 2000
GRID = N_TRANS // BLK


def _tc_dense_kernel(tx_ref, tpart_ref, cntt_ref, n_ref, mx_ref, mpart_ref,
                     cntm_ref,
                     wms_ref, wmnl_ref, wmnh_ref, bmi_ref,
                     wMs_ref, wMn_ref, bma_ref,
                     wct_ref, wcb_ref, bct_ref, wp_ref, bp_ref,
                     out_ref, g_scr):
    i = pl.program_id(0)

    @pl.when(i == 0)
    def _():
        ms = mpart_ref[0, :N_MACRO, :MACRO_IN]
        mc = jnp.maximum(cntm_ref[:N_MACRO], 1.0)
        m_agg = ms / mc
        h_macro = jnp.maximum(
            jnp.dot(mx_ref[...], wMs_ref[...], preferred_element_type=jnp.float32)
            + jnp.dot(m_agg, wMn_ref[...], preferred_element_type=jnp.float32)
            + bma_ref[...], 0.0)
        g_scr[...] = jnp.dot(h_macro, wcb_ref[...], preferred_element_type=jnp.float32)

    tc = jnp.maximum(cntt_ref[...], 1.0)
    t_agg_lo = tpart_ref[0] / tc
    t_agg_hi = tpart_ref[1] / tc
    h_micro = jnp.maximum(
        jnp.dot(tx_ref[...], wms_ref[...], preferred_element_type=jnp.float32)
        + jnp.dot(t_agg_lo, wmnl_ref[...], preferred_element_type=jnp.float32)
        + jnp.dot(t_agg_hi, wmnh_ref[...], preferred_element_type=jnp.float32)
        + bmi_ref[...], 0.0)

    cols = lax.broadcasted_iota(jnp.int32, (BLK, N_MACRO), 1)
    onehot = (n_ref[...] == cols).astype(jnp.float32)
    macro_per_trans = jnp.dot(onehot, g_scr[...], preferred_element_type=jnp.float32)

    h_final = jnp.maximum(
        jnp.dot(h_micro, wct_ref[...], preferred_element_type=jnp.float32)
        + macro_per_trans + bct_ref[...], 0.0)
    out_ref[...] = jnp.dot(h_final, wp_ref[...],
                           preferred_element_type=jnp.float32) + bp_ref[...]


def _tc_dense(tx, tpart, cntt2d, n2d, mx, mpart, cntm2d,
              wms, wmnl, wmnh, bmi, wMs, wMn, bma,
              wct, wcb, bct, wp, bp):
    whole = lambda shape: pl.BlockSpec(shape, lambda i: tuple(0 for _ in shape))
    return pl.pallas_call(
        _tc_dense_kernel,
        grid=(GRID,),
        in_specs=[
            pl.BlockSpec((BLK, TRANS_IN), lambda i: (i, 0)),
            pl.BlockSpec((NC, BLK, D_T), lambda i: (0, i, 0)),
            pl.BlockSpec((BLK, 1), lambda i: (i, 0)),
            pl.BlockSpec((BLK, 1), lambda i: (i, 0)),
            whole((N_MACRO, MACRO_IN)),
            whole((NC, ACC_M, D_M)),
            whole((ACC_M, 1)),
            whole((TRANS_IN, HID)),
            whole((HALF, HID)),
            whole((HALF, HID)),
            whole((1, HID)),
            whole((MACRO_IN, HID)),
            whole((MACRO_IN, HID)),
            whole((1, HID)),
            whole((HID, HID)),
            whole((HID, HID)),
            whole((1, HID)),
            whole((HID, 1)),
            whole((1, 1)),
        ],
        out_specs=pl.BlockSpec((BLK, 1), lambda i: (i, 0)),
        out_shape=jax.ShapeDtypeStruct((N_TRANS, 1), jnp.float32),
        scratch_shapes=[pltpu.VMEM((N_MACRO, HID), jnp.float32)],
    )(tx, tpart, cntt2d, n2d, mx, mpart, cntm2d, wms, wmnl, wmnh, bmi,
      wMs, wMn, bma, wct, wcb, bct, wp, bp)


def kernel(trans_x, macro_x, trans_edge_index, macro_edge_index, trans_to_neigh,
           W_micro_self, W_micro_neigh, b_micro,
           W_macro_self, W_macro_neigh, b_macro,
           W_cross_t, b_cross_t, W_cross_m, b_cross_m,
           W_pred, b_pred):
    f32 = jnp.float32

    t_tab0 = trans_x[:, :HALF]
    t_tab1 = trans_x[:, HALF:]
    m_tab = macro_x

    t_pad = E_T_PAD - E_TRANS
    m_pad = E_M_PAD - E_MACRO
    t_src = jnp.concatenate([trans_edge_index[0], jnp.zeros((t_pad,), jnp.int32)]
                            ).reshape(NS * T_CHUNKS, CHUNK)
    t_dst = jnp.concatenate([trans_edge_index[1], jnp.full((t_pad,), N_TRANS, jnp.int32)]
                            ).reshape(NS * T_CHUNKS, CHUNK)
    m_src = jnp.concatenate([macro_edge_index[0], jnp.zeros((m_pad,), jnp.int32)]
                            ).reshape(NS * M_CHUNKS, CHUNK)
    m_dst = jnp.concatenate([macro_edge_index[1], jnp.full((m_pad,), N_MACRO, jnp.int32)]
                            ).reshape(NS * M_CHUNKS, CHUNK)

    zt = jnp.zeros((ACC_T // NS, D_T), f32)
    zm = jnp.zeros((ACC_M // NS, D_M), f32)

    tpart, mpart, cntt, cntm = _sc_aggregate(t_tab0, t_tab1, m_tab, t_src, t_dst,
                                             m_src, m_dst, zt, zm)

    out2d = _tc_dense(
        trans_x, tpart, cntt.reshape(ACC_T, 1), trans_to_neigh.reshape(N_TRANS, 1),
        macro_x, mpart, cntm.reshape(ACC_M, 1),
        W_micro_self, W_micro_neigh[:HALF], W_micro_neigh[HALF:],
        b_micro.reshape(1, HID),
        W_macro_self, W_macro_neigh, b_macro.reshape(1, HID),
        W_cross_t[:HID], W_cross_t[HID:], b_cross_t.reshape(1, HID),
        W_pred, b_pred.reshape(1, 1))
    return out2d.squeeze(-1)
